# baseline (speedup 1.0000x reference)
import functools

import numpy as np

import jax
import jax.numpy as jnp
from jax import lax
from jax.experimental import pallas as pl
from jax.experimental.pallas import tpu as pltpu
from jax.experimental.pallas import tpu_sc as plsc


W58 = 58
NQ = 3368
BASE = 64
SQ = 3496
OFF3 = [BASE - (W58 + 1) + dy * W58 + dx for dy in range(3) for dx in range(3)]
SLAB = 3432
ND = 3248
SD = 3376
_VQ_CHUNKS = [(0, 424), (424, 424), (848, 424), (1272, 424), (1696, 424),
              (2120, 424), (2544, 424), (2968, 400)]
NQP = 3392
NBIN = 528


def _rup(n, m):
    return (n + m - 1) // m * m


def _bf(x):
    return x.astype(jnp.bfloat16)


def _np_qmask(n):
    p = np.arange(n)
    y, x = p // W58, p % W58
    ok = (y >= 1) & (y <= 56) & (x >= 1) & (x <= 56)
    return ok.astype(np.float32)[:, None]


_MASKQ = _np_qmask(NQ)
_MASKD = ((np.arange(ND) % W58) < 56).astype(np.float32)[:, None]


def _np_l1mask():
    m = np.zeros((4, _rup(57 * W58, 8), 1), np.float32)
    for ph in range(4):
        r, s = ph // 2, ph % 2
        p = np.arange(m.shape[1])
        u, v = p // W58, p % W58
        ok = (u <= 56) & (v <= 56)
        ok &= (u >= 1) if r == 0 else (u <= 55)
        ok &= (v >= 1) if s == 0 else (v <= 55)
        m[ph, :, 0] = ok.astype(np.float32)
    return m


_MASKL1 = _np_l1mask()


def _taps(ref, offsets, w_ref, n):
    acc = jnp.zeros((n, w_ref.shape[-1]), jnp.float32)
    for t, off in enumerate(offsets):
        acc = acc + jnp.dot(_bf(ref[0, pl.ds(off, n), :]), _bf(w_ref[t]),
                            preferred_element_type=jnp.float32)
    return acc


def _staps(ref, offsets, w_ref, n):
    acc = jnp.zeros((n, w_ref.shape[-1]), jnp.float32)
    for t, off in enumerate(offsets):
        acc = acc + jnp.dot(_bf(ref[pl.ds(off, n), :]), _bf(w_ref[t]),
                            preferred_element_type=jnp.float32)
    return acc


def _staps_relu(ref, offsets, w_ref, n):
    acc = jnp.zeros((n, w_ref.shape[-1]), jnp.float32)
    for t, off in enumerate(offsets):
        xt = jnp.maximum(ref[pl.ds(off, n), :], 0.0)
        acc = acc + jnp.dot(_bf(xt), _bf(w_ref[t]),
                            preferred_element_type=jnp.float32)
    return acc


def _store_frame(sref, val, first):
    C = val.shape[-1]
    sref[pl.ds(BASE, NQ), :] = _bf(val)

    @pl.when(first)
    def _():
        sref[pl.ds(0, BASE), :] = jnp.zeros((BASE, C), jnp.bfloat16)
        sref[pl.ds(BASE + NQ, SQ - BASE - NQ), :] = jnp.zeros(
            (SQ - BASE - NQ, C), jnp.bfloat16)


def _w9(w):
    return jnp.stack([w[:, :, dy, dx].T for dy in range(3) for dx in range(3)])


def _sc_hist(idx_flat):
    N = idx_flat.shape[0]
    per = N // 32
    mesh = plsc.VectorSubcoreMesh(core_axis_name="c", subcore_axis_name="s")

    @functools.partial(
        pl.kernel, mesh=mesh,
        out_type=jax.ShapeDtypeStruct((32, NBIN), jnp.float32),
        compiler_params=pltpu.CompilerParams(needs_layout_passes=False),
        scratch_types=[
            pltpu.VMEM((per,), jnp.int32),
            pltpu.VMEM((NBIN,), jnp.float32),
        ],
    )
    def k(idx_hbm, out_hbm, idx_v, bins_v):
        wid = lax.axis_index("s") * 2 + lax.axis_index("c")
        base = wid * per
        pltpu.sync_copy(idx_hbm.at[pl.ds(base, per)], idx_v)
        zeros16 = jnp.zeros((16,), jnp.float32)
        for j in range(NBIN // 16):
            bins_v[pl.ds(j * 16, 16)] = zeros16
        ones16 = jnp.full((16,), 1.0, jnp.float32)
        for j in range(per // 16):
            v = idx_v[pl.ds(j * 16, 16)]
            plsc.addupdate_scatter(bins_v, [v], ones16)
        pltpu.sync_copy(bins_v, out_hbm.at[wid])

    return k(idx_flat)


def _perp_body(total, h_ref, o_ref):
    counts = jnp.sum(h_ref[...], axis=0, keepdims=True)[:, :512]
    avg = counts / total
    t = jnp.sum(avg * jnp.log(avg + 1e-10), axis=1, keepdims=True)
    o_ref[...] = jnp.exp(-t)


def _l1_body(p_ref, w_ref, b_ref, m_ref, o_ref):
    n = p_ref.shape[-1]
    for ph in range(4):
        acc = jax.lax.dot_general(
            p_ref[0, ph], _bf(w_ref[...]), (((0,), (0,)), ((), ())),
            preferred_element_type=jnp.float32) + b_ref[...]
        acc = jnp.maximum(acc, 0.0)
        acc = acc * m_ref[ph]
        o_ref[0, ph, pl.ds(BASE, n), :] = _bf(acc)
        o_ref[0, ph, pl.ds(0, BASE), :] = jnp.zeros((BASE, acc.shape[1]),
                                                    jnp.bfloat16)
        tail = SLAB - BASE - n
        o_ref[0, ph, pl.ds(BASE + n, tail), :] = jnp.zeros(
            (tail, acc.shape[1]), jnp.bfloat16)


def _l1_call(patches, w1, b1, B, C1):
    npr = patches.shape[3]
    return pl.pallas_call(
        _l1_body,
        grid=(B,),
        in_specs=[
            pl.BlockSpec((1, 4, 48, npr), lambda i: (i, 0, 0, 0)),
            pl.BlockSpec((48, C1), lambda i: (0, 0)),
            pl.BlockSpec((1, C1), lambda i: (0, 0)),
            pl.BlockSpec((4, npr, 1), lambda i: (0, 0, 0)),
        ],
        out_specs=pl.BlockSpec((1, 4, SLAB, C1), lambda i: (i, 0, 0, 0)),
        out_shape=jax.ShapeDtypeStruct((B, 4, SLAB, C1), jnp.bfloat16),
    )(patches, w1, b1.reshape(1, C1), jnp.asarray(_MASKL1))


def _enc_body(total, slab_ref, w2_ref, b2_ref, w3_ref, b3_ref,
              r1a_ref, r1ab_ref, r1b_ref, r1bb_ref,
              r2a_ref, r2ab_ref, r2b_ref, r2bb_ref,
              pw_ref, pb_ref, cbt_ref, nz_ref, mq_ref,
              qf_ref, idx_ref, s1, s2):
    i = pl.program_id(0)
    first = i == 0
    offs2 = [ph * SLAB + BASE - (W58 + 1) + a * W58 + bb
             for ph in range(4) for a in range(2) for bb in range(2)]
    maskq = mq_ref[...]

    a2 = jnp.maximum(_taps(slab_ref, offs2, w2_ref, NQ) + b2_ref[...], 0.0)
    _store_frame(s1, a2 * maskq, first)
    a3 = _staps(s1, OFF3, w3_ref, NQ) + b3_ref[...]
    _store_frame(s2, a3 * maskq, first)
    h = jnp.maximum(_staps_relu(s2, OFF3, r1a_ref, NQ) + r1ab_ref[...], 0.0)
    y = s2[pl.ds(BASE, NQ), :] + jnp.dot(
        _bf(h), _bf(r1b_ref[...]), preferred_element_type=jnp.float32) \
        + r1bb_ref[...]
    _store_frame(s1, y * maskq, first)
    h = jnp.maximum(_staps_relu(s1, OFF3, r2a_ref, NQ) + r2ab_ref[...], 0.0)
    y = s1[pl.ds(BASE, NQ), :] + jnp.dot(
        _bf(h), _bf(r2b_ref[...]), preferred_element_type=jnp.float32) \
        + r2bb_ref[...]
    y = jnp.maximum(y, 0.0)
    _store_frame(s2, y * maskq, first)

    K = cbt_ref.shape[1]
    cbsq = jnp.sum(cbt_ref[...] * cbt_ref[...], axis=0, keepdims=True)
    for st, sz in _VQ_CHUNKS:
        zf = jnp.dot(s2[pl.ds(BASE + st, sz), :], _bf(pw_ref[...]),
                     preferred_element_type=jnp.float32) + pb_ref[...]
        sc = jnp.dot(_bf(zf), _bf(cbt_ref[...]),
                     preferred_element_type=jnp.float32)
        d2 = cbsq - 2.0 * sc
        m = jnp.min(d2, axis=1, keepdims=True)
        ii = jax.lax.broadcasted_iota(jnp.int32, (sz, K), 1)
        idx = jnp.min(jnp.where(d2 == m, ii, K), axis=1, keepdims=True)
        zsq = jnp.sum(zf * zf, axis=1, keepdims=True)
        nr = jnp.sqrt(jnp.maximum(m + zsq, 0.0))
        nz = nz_ref[0, pl.ds(st, sz), :].astype(jnp.float32)
        nv = jnp.sqrt(jnp.sum(nz * nz, axis=1, keepdims=True))
        mk = mq_ref[pl.ds(st, sz), :]
        qf_ref[0, pl.ds(BASE + st, sz), :] = _bf(
            (zf + (nr / (nv + 1e-12)) * nz) * mk)
        idx_ref[0, pl.ds(st, sz), :] = jnp.where(mk > 0.0, idx, K)
    D = pw_ref.shape[1]
    qf_ref[0, pl.ds(0, BASE), :] = jnp.zeros((BASE, D), jnp.bfloat16)
    qf_ref[0, pl.ds(BASE + NQ, SQ - BASE - NQ), :] = jnp.zeros(
        (SQ - BASE - NQ, D), jnp.bfloat16)
    idx_ref[0, pl.ds(NQ, NQP - NQ), :] = jnp.full((NQP - NQ, 1), K, jnp.int32)


def _dec_body(qf_ref, w1_ref, b1_ref,
              r1a_ref, r1ab_ref, r1b_ref, r1bb_ref,
              r2a_ref, r2ab_ref, r2b_ref, r2bb_ref,
              t1_ref, t1b_ref, t2_ref, t2b_ref, mq_ref, md_ref,
              o_ref, s1, s2, s4):
    i = pl.program_id(0)
    first = i == 0
    maskq = mq_ref[...]

    h1 = _taps(qf_ref, OFF3, w1_ref, NQ) + b1_ref[...]
    _store_frame(s1, h1 * maskq, first)
    h = jnp.maximum(_staps_relu(s1, OFF3, r1a_ref, NQ) + r1ab_ref[...], 0.0)
    y = s1[pl.ds(BASE, NQ), :] + jnp.dot(
        _bf(h), _bf(r1b_ref[...]), preferred_element_type=jnp.float32) \
        + r1bb_ref[...]
    _store_frame(s2, y * maskq, first)
    h = jnp.maximum(_staps_relu(s2, OFF3, r2a_ref, NQ) + r2ab_ref[...], 0.0)
    y = s2[pl.ds(BASE, NQ), :] + jnp.dot(
        _bf(h), _bf(r2b_ref[...]), preferred_element_type=jnp.float32) \
        + r2bb_ref[...]
    y = jnp.maximum(y, 0.0)
    _store_frame(s1, y * maskq, first)

    offs_d = [BASE + al * W58 + ga for al in range(3) for ga in range(3)]
    d1 = jnp.maximum(_staps(s1, offs_d, t1_ref, ND) + t1b_ref[...], 0.0)
    d1 = d1 * md_ref[...]
    s4[pl.ds(BASE, ND), :] = _bf(d1)

    @pl.when(first)
    def _():
        C = d1.shape[1]
        s4[pl.ds(0, BASE), :] = jnp.zeros((BASE, C), jnp.bfloat16)
        s4[pl.ds(BASE + ND, SD - BASE - ND), :] = jnp.zeros(
            (SD - BASE - ND, C), jnp.bfloat16)

    offs_d2 = [BASE + (dm - 1) * W58 + (dn - 1)
               for dm in range(3) for dn in range(3)]
    o_ref[0] = _staps(s4, offs_d2, t2_ref, ND) + t2b_ref[...]


def _deconv1_taps(w):
    Cin, Cout = w.shape[0], w.shape[1]
    zero = jnp.zeros((Cin, Cout), jnp.float32)

    def blk(al, ga, r, s):
        if r == 0:
            if al > 1:
                return zero
            ky = 2 * al
        else:
            if al < 1:
                return zero
            ky = 2 * al - 1
        if s == 0:
            if ga > 1:
                return zero
            kx = 2 * ga
        else:
            if ga < 1:
                return zero
            kx = 2 * ga - 1
        return w[:, :, ky, kx]

    return jnp.stack([
        jnp.concatenate([blk(al, ga, r, s)
                         for r in range(2) for s in range(2)], axis=1)
        for al in range(3) for ga in range(3)])


def _deconv2_taps(w):
    Cin, Cout = w.shape[0], w.shape[1]
    W2 = jnp.zeros((9, 4 * Cin, 4 * 4 * Cout), jnp.float32)
    for rho in range(2):
        for sig in range(2):
            for r2 in range(2):
                for s2 in range(2):
                    col = ((rho * 2 + sig) * 4 + r2 * 2 + s2) * Cout
                    als = (0, 1) if r2 == 0 else (1, 2)
                    gas = (0, 1) if s2 == 0 else (1, 2)
                    for al in als:
                        ky = 2 * al if r2 == 0 else 2 * al - 1
                        dm = (rho + al - 1) // 2
                        r = (rho + al - 1) % 2
                        for ga in gas:
                            kx = 2 * ga if s2 == 0 else 2 * ga - 1
                            dn = (sig + ga - 1) // 2
                            s = (sig + ga - 1) % 2
                            t = (dm + 1) * 3 + (dn + 1)
                            rowb = (r * 2 + s) * Cin
                            W2 = W2.at[t, rowb:rowb + Cin,
                                       col:col + Cout].add(w[:, :, ky, kx])
    return W2


def kernel(x, noise, enc_w1, enc_b1, enc_w2, enc_b2, enc_w3, enc_b3,
           enc_r1_w1, enc_r1_b1, enc_r1_w2, enc_r1_b2,
           enc_r2_w1, enc_r2_b1, enc_r2_w2, enc_r2_b2,
           pre_w, pre_b, codebook,
           dec_w1, dec_b1, dec_r1_w1, dec_r1_b1, dec_r1_w2, dec_r1_b2,
           dec_r2_w1, dec_r2_b1, dec_r2_w2, dec_r2_b2,
           dec_tw1, dec_tb1, dec_tw2, dec_tb2):
    B = x.shape[0]
    C1 = enc_w1.shape[0]
    Hc = enc_w2.shape[0]
    D = pre_w.shape[0]
    K = codebook.shape[0]
    npr = _rup(57 * W58, 8)
    xp4 = jnp.pad(x, ((0, 0), (0, 0), (3, 5), (3, 5))).astype(jnp.bfloat16)
    xa = xp4.reshape(B, 3, 58, 4, 232).transpose(0, 1, 3, 4, 2)
    xc = xa.reshape(B, 3, 4, 58, 4, 58)
    phs = []
    for r in range(2):
        for s in range(2):
            planes = []
            for ky in range(4):
                oy = 2 * r + ky
                py, q0 = oy % 4, oy // 4
                for kx in range(4):
                    ox = 2 * s + kx
                    pc, cg0 = ox % 4, ox // 4
                    for c in range(3):
                        planes.append(
                            xc[:, c, py, cg0:cg0 + 57, pc, q0:q0 + 57])
            phs.append(jnp.stack(planes, axis=1))
    pat = jnp.stack(phs, axis=1)
    pat = pat.transpose(0, 1, 2, 4, 3)
    pat = jnp.pad(pat, ((0, 0), (0, 0), (0, 0), (0, 0), (0, 1)))
    patches = jnp.pad(pat.reshape(B, 4, 48, 57 * W58),
                      ((0, 0), (0, 0), (0, 0), (0, npr - 57 * W58)))
    w1 = jnp.transpose(enc_w1, (2, 3, 1, 0)).reshape(48, C1)
    slabs = _l1_call(patches, w1, enc_b1, B, C1)
    slabs = slabs.reshape(B, 4 * SLAB, C1)

    nz = noise.reshape(B, 56, 56, D).astype(jnp.bfloat16)
    nz = jnp.pad(nz, ((0, 0), (1, 1), (1, 1), (0, 0))).reshape(B, 58 * 58, D)
    nz = jnp.pad(nz, ((0, 0), (0, NQ - 58 * 58), (0, 0)))

    w2t = jnp.stack([enc_w2[:, :, 2 * a + r, 2 * bb + s].T
                     for r in range(2) for s in range(2)
                     for a in range(2) for bb in range(2)])
    total = float(B * 56 * 56)

    qf, idx_out = pl.pallas_call(
        lambda *refs: _enc_body(total, *refs),
        grid=(B,),
        in_specs=[
            pl.BlockSpec((1, 4 * SLAB, C1), lambda i: (i, 0, 0)),
            pl.BlockSpec((16, C1, Hc), lambda i: (0, 0, 0)),
            pl.BlockSpec((1, Hc), lambda i: (0, 0)),
            pl.BlockSpec((9, Hc, Hc), lambda i: (0, 0, 0)),
            pl.BlockSpec((1, Hc), lambda i: (0, 0)),
            pl.BlockSpec((9, Hc, 32), lambda i: (0, 0, 0)),
            pl.BlockSpec((1, 32), lambda i: (0, 0)),
            pl.BlockSpec((32, Hc), lambda i: (0, 0)),
            pl.BlockSpec((1, Hc), lambda i: (0, 0)),
            pl.BlockSpec((9, Hc, 32), lambda i: (0, 0, 0)),
            pl.BlockSpec((1, 32), lambda i: (0, 0)),
            pl.BlockSpec((32, Hc), lambda i: (0, 0)),
            pl.BlockSpec((1, Hc), lambda i: (0, 0)),
            pl.BlockSpec((Hc, D), lambda i: (0, 0)),
            pl.BlockSpec((1, D), lambda i: (0, 0)),
            pl.BlockSpec((D, K), lambda i: (0, 0)),
            pl.BlockSpec((1, NQ, D), lambda i: (i, 0, 0)),
            pl.BlockSpec((NQ, 1), lambda i: (0, 0)),
        ],
        out_specs=[
            pl.BlockSpec((1, SQ, D), lambda i: (i, 0, 0)),
            pl.BlockSpec((1, NQP, 1), lambda i: (i, 0, 0)),
        ],
        out_shape=[
            jax.ShapeDtypeStruct((B, SQ, D), jnp.bfloat16),
            jax.ShapeDtypeStruct((B, NQP, 1), jnp.int32),
        ],
        scratch_shapes=[
            pltpu.VMEM((SQ, Hc), jnp.bfloat16),
            pltpu.VMEM((SQ, Hc), jnp.bfloat16),
        ],
    )(slabs, w2t, enc_b2.reshape(1, Hc), _w9(enc_w3), enc_b3.reshape(1, Hc),
      _w9(enc_r1_w1), enc_r1_b1.reshape(1, 32),
      enc_r1_w2[:, :, 0, 0].T, enc_r1_b2.reshape(1, Hc),
      _w9(enc_r2_w1), enc_r2_b1.reshape(1, 32),
      enc_r2_w2[:, :, 0, 0].T, enc_r2_b2.reshape(1, Hc),
      pre_w[:, :, 0, 0].T, pre_b.reshape(1, D), codebook.T, nz,
      jnp.asarray(_MASKQ))

    parts = _sc_hist(idx_out.reshape(B * NQP))
    perp = pl.pallas_call(
        lambda h_ref, o_ref: _perp_body(total, h_ref, o_ref),
        grid=(1,),
        in_specs=[pl.BlockSpec((32, NBIN), lambda i: (0, 0))],
        out_specs=pl.BlockSpec((1, 1), lambda i: (0, 0)),
        out_shape=jax.ShapeDtypeStruct((1, 1), jnp.float32),
    )(parts)

    out = pl.pallas_call(
        _dec_body,
        grid=(B,),
        in_specs=[
            pl.BlockSpec((1, SQ, D), lambda i: (i, 0, 0)),
            pl.BlockSpec((9, D, Hc), lambda i: (0, 0, 0)),
            pl.BlockSpec((1, Hc), lambda i: (0, 0)),
            pl.BlockSpec((9, Hc, 32), lambda i: (0, 0, 0)),
            pl.BlockSpec((1, 32), lambda i: (0, 0)),
            pl.BlockSpec((32, Hc), lambda i: (0, 0)),
            pl.BlockSpec((1, Hc), lambda i: (0, 0)),
            pl.BlockSpec((9, Hc, 32), lambda i: (0, 0, 0)),
            pl.BlockSpec((1, 32), lambda i: (0, 0)),
            pl.BlockSpec((32, Hc), lambda i: (0, 0)),
            pl.BlockSpec((1, Hc), lambda i: (0, 0)),
            pl.BlockSpec((9, Hc, 4 * C1), lambda i: (0, 0, 0)),
            pl.BlockSpec((1, 4 * C1), lambda i: (0, 0)),
            pl.BlockSpec((9, 4 * C1, 48), lambda i: (0, 0, 0)),
            pl.BlockSpec((1, 48), lambda i: (0, 0)),
            pl.BlockSpec((NQ, 1), lambda i: (0, 0)),
            pl.BlockSpec((ND, 1), lambda i: (0, 0)),
        ],
        out_specs=pl.BlockSpec((1, ND, 48), lambda i: (i, 0, 0)),
        out_shape=jax.ShapeDtypeStruct((B, ND, 48), jnp.float32),
        scratch_shapes=[
            pltpu.VMEM((SQ, Hc), jnp.bfloat16),
            pltpu.VMEM((SQ, Hc), jnp.bfloat16),
            pltpu.VMEM((SD, 4 * C1), jnp.bfloat16),
        ],
    )(qf, _w9(dec_w1), dec_b1.reshape(1, Hc),
      _w9(dec_r1_w1), dec_r1_b1.reshape(1, 32),
      dec_r1_w2[:, :, 0, 0].T, dec_r1_b2.reshape(1, Hc),
      _w9(dec_r2_w1), dec_r2_b1.reshape(1, 32),
      dec_r2_w2[:, :, 0, 0].T, dec_r2_b2.reshape(1, Hc),
      _deconv1_taps(dec_tw1), jnp.tile(dec_tb1, 4).reshape(1, 4 * C1),
      _deconv2_taps(dec_tw2), jnp.tile(dec_tb2, 16).reshape(1, 48),
      jnp.asarray(_MASKQ), jnp.asarray(_MASKD))

    xr = out.reshape(B, 56, W58, 2, 2, 2, 2, 3)[:, :, :56]
    xr = xr.transpose(0, 7, 1, 3, 5, 2, 4, 6)
    x_recon = xr.reshape(B, 3, 224, 224)
    return (x_recon, perp.reshape(()))
